# baseline (speedup 1.0000x reference)
import functools

import jax
import jax.numpy as jnp
from jax.experimental import pallas as pl
from jax.experimental.pallas import tpu as pltpu


def _gate_kernel(x_ref, gw_ref, gb_ref, p_ref, *, bt, ne):
    logits = jnp.dot(x_ref[...], gw_ref[...],
                     preferred_element_type=jnp.float32) + gb_ref[...]
    iota = jax.lax.broadcasted_iota(jnp.int32, (bt, ne), 1)
    m1 = jnp.max(logits, axis=1, keepdims=True)
    i1 = jnp.min(jnp.where(logits == m1, iota, ne), axis=1, keepdims=True)
    f1 = iota == i1
    l2 = jnp.where(f1, -jnp.inf, logits)
    m2 = jnp.max(l2, axis=1, keepdims=True)
    i2 = jnp.min(jnp.where(l2 == m2, iota, ne), axis=1, keepdims=True)
    f2 = iota == i2
    p1 = 1.0 / (1.0 + jnp.exp(m2 - m1))
    p2 = 1.0 - p1
    p_ref[...] = p1 * f1.astype(jnp.float32) + p2 * f2.astype(jnp.float32)


def _expert_kernel(p_ref, xe_ref, w_ref, b_ref, out_ref, *, ne):
    probs = p_ref[...]
    acc = jnp.dot(probs, b_ref[...], preferred_element_type=jnp.float32)
    xe = xe_ref[...]
    for e in range(ne):
        y = jnp.dot(xe, w_ref[e], preferred_element_type=jnp.float32)
        acc += probs[:, e:e + 1] * y
    out_ref[...] = acc


def kernel(insample_y, gate_W, gate_b, expert_W, expert_b):
    n_tok, d_model = insample_y.shape
    n_experts, _, out_features = expert_W.shape
    bt = min(2048, n_tok)
    bn = 256

    x = insample_y
    xe = insample_y.astype(jnp.bfloat16)
    ew = expert_W.astype(jnp.bfloat16)
    gb2 = gate_b.reshape(1, n_experts)

    gate_fn = functools.partial(_gate_kernel, bt=bt, ne=n_experts)
    probs = pl.pallas_call(
        gate_fn,
        grid=(n_tok // bt,),
        in_specs=[
            pl.BlockSpec((bt, d_model), lambda t: (t, 0)),
            pl.BlockSpec((d_model, n_experts), lambda t: (0, 0)),
            pl.BlockSpec((1, n_experts), lambda t: (0, 0)),
        ],
        out_specs=pl.BlockSpec((bt, n_experts), lambda t: (t, 0)),
        out_shape=jax.ShapeDtypeStruct((n_tok, n_experts), jnp.float32),
    )(x, gate_W, gb2)

    mm_fn = functools.partial(_expert_kernel, ne=n_experts)
    return pl.pallas_call(
        mm_fn,
        grid=(out_features // bn, n_tok // bt),
        in_specs=[
            pl.BlockSpec((bt, n_experts), lambda n, t: (t, 0)),
            pl.BlockSpec((bt, d_model), lambda n, t: (t, 0)),
            pl.BlockSpec((n_experts, d_model, bn), lambda n, t: (0, 0, n)),
            pl.BlockSpec((n_experts, bn), lambda n, t: (0, n)),
        ],
        out_specs=pl.BlockSpec((bt, bn), lambda n, t: (t, n)),
        out_shape=jax.ShapeDtypeStruct((n_tok, out_features), jnp.float32),
        compiler_params=pltpu.CompilerParams(
            dimension_semantics=("parallel", "parallel")),
    )(probs, xe, ew, expert_b)


# speedup vs baseline: 1.0344x; 1.0344x over previous
import functools

import jax
import jax.numpy as jnp
from jax.experimental import pallas as pl
from jax.experimental.pallas import tpu as pltpu


def _gate_kernel(x_ref, gw_ref, gb_ref, p_ref, xe_ref, *, bt, ne):
    xe_ref[...] = x_ref[...].astype(jnp.bfloat16)
    logits = jnp.dot(x_ref[...], gw_ref[...],
                     preferred_element_type=jnp.float32) + gb_ref[...]
    iota = jax.lax.broadcasted_iota(jnp.int32, (bt, ne), 1)
    m1 = jnp.max(logits, axis=1, keepdims=True)
    i1 = jnp.min(jnp.where(logits == m1, iota, ne), axis=1, keepdims=True)
    f1 = iota == i1
    l2 = jnp.where(f1, -jnp.inf, logits)
    m2 = jnp.max(l2, axis=1, keepdims=True)
    i2 = jnp.min(jnp.where(l2 == m2, iota, ne), axis=1, keepdims=True)
    f2 = iota == i2
    p1 = 1.0 / (1.0 + jnp.exp(m2 - m1))
    p2 = 1.0 - p1
    p_ref[...] = p1 * f1.astype(jnp.float32) + p2 * f2.astype(jnp.float32)


def _expert_kernel(p_ref, xe_ref, w_ref, b_ref, out_ref, *, ne):
    probs = p_ref[...]
    acc = jnp.dot(probs, b_ref[...], preferred_element_type=jnp.float32)
    xe = xe_ref[...]
    for e in range(ne):
        y = jnp.dot(xe, w_ref[e], preferred_element_type=jnp.float32)
        acc += probs[:, e:e + 1] * y
    out_ref[...] = acc


def kernel(insample_y, gate_W, gate_b, expert_W, expert_b):
    n_tok, d_model = insample_y.shape
    n_experts, _, out_features = expert_W.shape
    bt = min(2048, n_tok)
    bn = 256

    x = insample_y
    ew = expert_W.astype(jnp.bfloat16)
    gb2 = gate_b.reshape(1, n_experts)

    gate_fn = functools.partial(_gate_kernel, bt=bt, ne=n_experts)
    probs, xe = pl.pallas_call(
        gate_fn,
        grid=(n_tok // bt,),
        in_specs=[
            pl.BlockSpec((bt, d_model), lambda t: (t, 0)),
            pl.BlockSpec((d_model, n_experts), lambda t: (0, 0)),
            pl.BlockSpec((1, n_experts), lambda t: (0, 0)),
        ],
        out_specs=[
            pl.BlockSpec((bt, n_experts), lambda t: (t, 0)),
            pl.BlockSpec((bt, d_model), lambda t: (t, 0)),
        ],
        out_shape=[
            jax.ShapeDtypeStruct((n_tok, n_experts), jnp.float32),
            jax.ShapeDtypeStruct((n_tok, d_model), jnp.bfloat16),
        ],
    )(x, gate_W, gb2)

    mm_fn = functools.partial(_expert_kernel, ne=n_experts)
    return pl.pallas_call(
        mm_fn,
        grid=(out_features // bn, n_tok // bt),
        in_specs=[
            pl.BlockSpec((bt, n_experts), lambda n, t: (t, 0)),
            pl.BlockSpec((bt, d_model), lambda n, t: (t, 0)),
            pl.BlockSpec((n_experts, d_model, bn), lambda n, t: (0, 0, n)),
            pl.BlockSpec((n_experts, bn), lambda n, t: (0, n)),
        ],
        out_specs=pl.BlockSpec((bt, bn), lambda n, t: (t, n)),
        out_shape=jax.ShapeDtypeStruct((n_tok, out_features), jnp.float32),
        compiler_params=pltpu.CompilerParams(
            dimension_semantics=("parallel", "parallel")),
    )(probs, xe, ew, expert_b)
